# Initial kernel scaffold; baseline (speedup 1.0000x reference)
import jax
import jax.numpy as jnp
from jax.experimental import pallas as pl


def kernel(vmem, vlastmem, labels):
    raise NotImplementedError("write your pallas kernel here")


# speedup vs baseline: 61.2817x; 61.2817x over previous
import functools

import jax
import jax.numpy as jnp
from jax import lax
from jax.experimental import pallas as pl
from jax.experimental.pallas import tpu as pltpu
from jax.experimental.pallas import tpu_sc as plsc

_C = 5
_B, _N, _T = 32, 64, 256
_L = 16
_NCHUNK = _T // _L
_MAXCL = 48
_BIGI = 1 << 30
_NEGF = -3.0e38


def _sc_body(vmem_hbm, labels_hbm, spike_hbm, part_hbm,
             v_v, lab_v, cnt_v, sp_v, spk_v, pv_v):
    wid = lax.axis_index("s") * 2 + lax.axis_index("c")
    pltpu.sync_copy(vmem_hbm.at[wid], v_v)
    pltpu.sync_copy(labels_hbm, lab_v)

    iota = lax.iota(jnp.int32, _L)
    tk = jnp.maximum(iota - 1, 0)
    ones_i = jnp.where(iota >= 0, jnp.int32(1), jnp.int32(0))
    lane0 = iota == 0

    labv = plsc.load_gather(lab_v, [jnp.full((_L,), wid, jnp.int32)])
    lab_s = jnp.max(labv)

    def row_body(n, loss):
        zer = jnp.where(iota < 0, jnp.int32(1), jnp.int32(0))
        for t in range(_MAXCL // _L):
            cnt_v[pl.ds(t * _L, _L)] = zer
        carry_cm = jnp.int32(-1)
        carry_nc = jnp.int32(0)
        row_max = jnp.float32(_NEGF)
        for k in range(_NCHUNK):
            vv = v_v[n, pl.ds(k * _L, _L)]
            idx_k = iota + (k * _L)
            sb = vv >= 0.0
            posv = jnp.where(sb, idx_k, jnp.int32(-1))
            cml = plsc.cummax(posv)
            sh = jnp.take_along_axis(cml, tk, axis=0)
            prev = jnp.maximum(jnp.where(iota >= 1, sh, jnp.int32(-1)),
                               carry_cm)
            newb = sb & ((prev < 0) | ((idx_k - prev) > _C))
            ncl = plsc.cumsum(jnp.where(newb, jnp.int32(1), jnp.int32(0)))
            r = ncl - 1 + carry_nc
            plsc.addupdate_scatter(cnt_v, [r], ones_i, mask=sb)
            plsc.store_scatter(sp_v, [r], idx_k, mask=newb)
            carry_cm = jnp.maximum(carry_cm, jnp.max(posv))
            carry_nc = carry_nc + jnp.max(ncl)
            row_max = jnp.maximum(row_max, jnp.max(vv))
        m_cl = carry_nc
        kmin = jnp.int32(_BIGI)
        for t in range(_MAXCL // _L):
            ct = cnt_v[pl.ds(t * _L, _L)]
            lane = iota + (t * _L)
            key = jnp.where(lane < m_cl, ct * 512 + lane, jnp.int32(_BIGI))
            kmin = jnp.minimum(kmin, jnp.min(key))
        ic = jnp.bitwise_and(kmin, jnp.int32(511))
        icc = jnp.minimum(ic, jnp.int32(_MAXCL - 1))
        bv = plsc.load_gather(sp_v, [jnp.full((_L,), icc, jnp.int32)])
        nv = plsc.load_gather(
            sp_v, [jnp.full((_L,), jnp.minimum(icc + 1, _MAXCL - 1),
                            jnp.int32)])
        b_s = jnp.where(m_cl > 0, jnp.max(bv), jnp.int32(_T + 1))
        nb_s = jnp.where(ic + 1 < m_cl, jnp.max(nv), jnp.int32(_T))
        tu = jnp.float32(_NEGF)
        for k in range(_NCHUNK):
            vv = v_v[n, pl.ds(k * _L, _L)]
            idx_k = iota + (k * _L)
            span = (idx_k >= b_s) & (idx_k < nb_s)
            tu = jnp.maximum(tu, jnp.max(jnp.where(span, vv, jnp.float32(_NEGF))))
        ltf = jnp.where(lab_s == n, jnp.float32(1.0), jnp.float32(0.0))
        mf = m_cl.astype(jnp.float32)
        contrib = (jnp.where(ltf > mf, -row_max, jnp.float32(0.0))
                   + jnp.where(ltf < mf, tu, jnp.float32(0.0)))
        plsc.store_scatter(spk_v, [jnp.full((_L,), n, jnp.int32)],
                           jnp.full((_L,), mf, jnp.float32), mask=lane0)
        return loss + contrib

    loss = lax.fori_loop(0, _N, row_body, jnp.float32(0.0))

    pltpu.sync_copy(spk_v, spike_hbm.at[wid])
    pv_v[...] = jnp.where(lane0, loss, jnp.float32(0.0))
    pltpu.sync_copy(pv_v, part_hbm.at[wid])


@jax.jit
def _sc_call(vmem, labels):
    mesh = plsc.VectorSubcoreMesh(core_axis_name="c", subcore_axis_name="s")
    fn = pl.kernel(
        _sc_body,
        out_type=(
            jax.ShapeDtypeStruct((_B, _N), jnp.float32),
            jax.ShapeDtypeStruct((_B, _L), jnp.float32),
        ),
        mesh=mesh,
        compiler_params=pltpu.CompilerParams(needs_layout_passes=False),
        scratch_types=[
            pltpu.VMEM((_N, _T), jnp.float32),
            pltpu.VMEM((_B,), jnp.int32),
            pltpu.VMEM((_MAXCL,), jnp.int32),
            pltpu.VMEM((_MAXCL,), jnp.int32),
            pltpu.VMEM((_N,), jnp.float32),
            pltpu.VMEM((_L,), jnp.float32),
        ],
    )
    return fn(vmem, labels)


def _tc_sum(part):
    def body(p_ref, o_ref):
        o_ref[0, 0] = jnp.sum(p_ref[...])

    return pl.pallas_call(
        body,
        out_shape=jax.ShapeDtypeStruct((1, 1), jnp.float32),
        out_specs=pl.BlockSpec(memory_space=pltpu.SMEM),
    )(part)


def kernel(vmem, vlastmem, labels):
    del vlastmem
    spike, part = _sc_call(vmem, labels.astype(jnp.int32))
    loss = _tc_sum(part).reshape(())
    return loss, spike
